# Initial kernel scaffold; baseline (speedup 1.0000x reference)
import functools

import jax
import jax.numpy as jnp
from jax import lax
from jax.experimental import pallas as pl
from jax.experimental.pallas import tpu as pltpu
from jax.experimental.pallas import tpu_sc as plsc

N = 10000
E = 320000
NFEAT = 128
NHID = 64
NOUT = 32

N2 = 2 * N
NE2 = 2 * E
CHUNK = 128
NCORES = 2
NSUB = 16
NW = NCORES * NSUB
ROWS = NE2 // CHUNK
RPW = -(-ROWS // NW)
RPW += RPW % 2
ROWS_PAD = RPW * NW
E_PAD = ROWS_PAD * CHUNK
SINK = N2
NP = 20016
ZPT = NP // NSUB
OPT = N2 // NSUB


def _make_spmm(width):
    mesh = plsc.VectorSubcoreMesh(
        core_axis_name="c", subcore_axis_name="s",
        num_cores=NCORES, num_subcores=NSUB)

    @functools.partial(
        pl.kernel,
        out_type=jax.ShapeDtypeStruct((NCORES, N2, width), jnp.float32),
        mesh=mesh,
        scratch_types=[
            pltpu.VMEM((RPW, CHUNK), jnp.int32),
            pltpu.VMEM((RPW, CHUNK), jnp.int32),
            pltpu.VMEM((CHUNK, width), jnp.float32),
            pltpu.VMEM_SHARED((NP, width), jnp.float32),
            pltpu.SemaphoreType.DMA,
        ],
    )
    def spmm(table_hbm, src_hbm, dst_hbm, zeros_hbm, out_hbm,
             src_v, dst_v, buf, acc, sem):
        cid = lax.axis_index("c")
        sid = lax.axis_index("s")
        wid = sid * NCORES + cid
        pltpu.sync_copy(zeros_hbm.at[pl.ds(sid * ZPT, ZPT)],
                        acc.at[pl.ds(sid * ZPT, ZPT)])
        pltpu.sync_copy(src_hbm.at[pl.ds(wid * RPW, RPW)], src_v)
        pltpu.sync_copy(dst_hbm.at[pl.ds(wid * RPW, RPW)], dst_v)
        plsc.subcore_barrier()

        @pl.loop(0, RPW)
        def _(j):
            pltpu.async_copy(table_hbm.at[src_v.at[j]], buf, sem).wait()
            pltpu.sync_copy(buf, acc.at[dst_v.at[j]], add=True)

        plsc.subcore_barrier()
        pltpu.sync_copy(acc.at[pl.ds(sid * OPT, OPT)],
                        out_hbm.at[cid, pl.ds(sid * OPT, OPT)])

    return spmm


_spmm64 = _make_spmm(NHID)
_spmm32 = _make_spmm(NOUT)


_MMBLK = 2000


def _mm1_body(x_ref, w_ref, o_ref):
    o_ref[...] = jnp.dot(x_ref[...], w_ref[...],
                         preferred_element_type=jnp.float32)


def _mm1(x, w1):
    return pl.pallas_call(
        _mm1_body,
        grid=(N2 // _MMBLK,),
        in_specs=[
            pl.BlockSpec((_MMBLK, NFEAT), lambda i: (i, 0)),
            pl.BlockSpec((NFEAT, NHID), lambda i: (0, 0)),
        ],
        out_specs=pl.BlockSpec((_MMBLK, NHID), lambda i: (i, 0)),
        out_shape=jax.ShapeDtypeStruct((N2, NHID), jnp.float32),
    )(x, w1)


def _fused2_body(p_ref, b1_ref, w2_ref, o_ref):
    a = p_ref[0] + p_ref[1] + b1_ref[...]
    h = jnp.maximum(a, 0.0)
    o_ref[...] = jnp.dot(h, w2_ref[...], preferred_element_type=jnp.float32)


def _fused2(parts, b1, w2):
    return pl.pallas_call(
        _fused2_body,
        grid=(N2 // _MMBLK,),
        in_specs=[
            pl.BlockSpec((NCORES, _MMBLK, NHID), lambda i: (0, i, 0)),
            pl.BlockSpec((1, NHID), lambda i: (0, 0)),
            pl.BlockSpec((NHID, NOUT), lambda i: (0, 0)),
        ],
        out_specs=pl.BlockSpec((_MMBLK, NOUT), lambda i: (i, 0)),
        out_shape=jax.ShapeDtypeStruct((N2, NOUT), jnp.float32),
    )(parts, b1, w2)


def _final_body(p_ref, b2_ref, o_ref):
    s = p_ref[0] + p_ref[1] + b2_ref[...]
    d = s[:N] - s[N:] + 1e-6
    o_ref[...] = jnp.sqrt(jnp.sum(d * d, axis=1))


def _final(parts, b2):
    return pl.pallas_call(
        _final_body,
        out_shape=jax.ShapeDtypeStruct((N,), jnp.float32),
    )(parts, b2)


def kernel(x1, adj1, x2, adj2, W1, b1, W2, b2):
    x = jnp.concatenate([x1, x2], axis=0)
    src = jnp.concatenate([adj1[0], adj2[0] + N])
    dst = jnp.concatenate([adj1[1], adj2[1] + N])
    pad = E_PAD - NE2
    src = jnp.concatenate([src, jnp.zeros((pad,), jnp.int32)])
    dst = jnp.concatenate([dst, jnp.full((pad,), SINK, jnp.int32)])
    src2d = src.reshape(ROWS_PAD, CHUNK)
    dst2d = dst.reshape(ROWS_PAD, CHUNK)
    zeros64 = jnp.zeros((NP, NHID), jnp.float32)
    zeros32 = jnp.zeros((NP, NOUT), jnp.float32)

    support = _mm1(x, W1)
    agg1 = _spmm64(support, src2d, dst2d, zeros64)
    support2 = _fused2(agg1, b1.reshape(1, NHID), W2)
    agg2 = _spmm32(support2, src2d, dst2d, zeros32)
    return _final(agg2, b2.reshape(1, NOUT))


# speedup vs baseline: 4.5194x; 4.5194x over previous
import functools

import jax
import jax.numpy as jnp
from jax import lax
from jax.experimental import pallas as pl
from jax.experimental.pallas import tpu as pltpu
from jax.experimental.pallas import tpu_sc as plsc

N = 10000
E = 320000
NFEAT = 128
NHID = 64
NOUT = 32

N2 = 2 * N
NE2 = 2 * E
CHUNK = 128
NCORES = 2
NSUB = 16
NW = NCORES * NSUB
ROWS = NE2 // CHUNK
RPW = -(-ROWS // NW)
RPW = -(-RPW // 8) * 8
ROWS_PAD = RPW * NW
E_PAD = ROWS_PAD * CHUNK
SINK = N2
ZPT = 1256
NP = ZPT * NSUB


def _make_spmm(width):
    mesh = plsc.VectorSubcoreMesh(
        core_axis_name="c", subcore_axis_name="s",
        num_cores=NCORES, num_subcores=NSUB)

    @functools.partial(
        pl.kernel,
        out_type=jax.ShapeDtypeStruct((NCORES, NP, width), jnp.float32),
        mesh=mesh,
        scratch_types=[
            pltpu.VMEM((RPW, CHUNK), jnp.int32),
            pltpu.VMEM((RPW, CHUNK), jnp.int32),
            pltpu.VMEM((CHUNK, width), jnp.float32),
            pltpu.VMEM_SHARED((NP, width), jnp.float32),
            pltpu.SemaphoreType.DMA,
        ],
        compiler_params=pltpu.CompilerParams(use_tc_tiling_on_sc=False),
    )
    def spmm(table_hbm, src_hbm, dst_hbm, zeros_hbm, out_hbm,
             src_v, dst_v, buf, acc, sem):
        cid = lax.axis_index("c")
        sid = lax.axis_index("s")
        wid = sid * NCORES + cid
        pltpu.sync_copy(zeros_hbm.at[pl.ds(sid * ZPT, ZPT)],
                        acc.at[pl.ds(sid * ZPT, ZPT)])
        pltpu.sync_copy(src_hbm.at[pl.ds(wid * RPW, RPW)], src_v)
        pltpu.sync_copy(dst_hbm.at[pl.ds(wid * RPW, RPW)], dst_v)
        plsc.subcore_barrier()

        @pl.loop(0, RPW)
        def _(j):
            pltpu.async_copy(table_hbm.at[src_v.at[j]], buf, sem).wait()
            pltpu.sync_copy(buf, acc.at[dst_v.at[j]], add=True)

        plsc.subcore_barrier()
        pltpu.sync_copy(acc.at[pl.ds(sid * ZPT, ZPT)],
                        out_hbm.at[cid, pl.ds(sid * ZPT, ZPT)])

    return spmm


_spmm_cache = {}


def _spmm(width):
    if width not in _spmm_cache:
        _spmm_cache[width] = _make_spmm(width)
    return _spmm_cache[width]


_MMBLK = 2000


def _mm1_body(x_ref, w_ref, o_ref):
    o_ref[...] = jnp.dot(x_ref[...], w_ref[...],
                         preferred_element_type=jnp.float32)


def _mm1(x, w1):
    return pl.pallas_call(
        _mm1_body,
        grid=(N2 // _MMBLK,),
        in_specs=[
            pl.BlockSpec((_MMBLK, NFEAT), lambda i: (i, 0)),
            pl.BlockSpec((NFEAT, NHID), lambda i: (0, 0)),
        ],
        out_specs=pl.BlockSpec((_MMBLK, NHID), lambda i: (i, 0)),
        out_shape=jax.ShapeDtypeStruct((N2, NHID), jnp.float32),
    )(x, w1)


def _fused2_body(p0_ref, p1_ref, b1_ref, w2_ref, o_ref):
    a = p0_ref[0] + p1_ref[0] + b1_ref[...]
    h = jnp.maximum(a, 0.0)
    o_ref[...] = jnp.dot(h, w2_ref[...], preferred_element_type=jnp.float32)


def _fused2(parts, b1, w2):
    return pl.pallas_call(
        _fused2_body,
        grid=(N2 // _MMBLK,),
        in_specs=[
            pl.BlockSpec((1, _MMBLK, NHID), lambda i: (0, i, 0)),
            pl.BlockSpec((1, _MMBLK, NHID), lambda i: (1, i, 0)),
            pl.BlockSpec((1, NHID), lambda i: (0, 0)),
            pl.BlockSpec((NHID, NOUT), lambda i: (0, 0)),
        ],
        out_specs=pl.BlockSpec((_MMBLK, NOUT), lambda i: (i, 0)),
        out_shape=jax.ShapeDtypeStruct((N2, NOUT), jnp.float32),
    )(parts, parts, b1, w2)


def _final_body(p_ref, b2_ref, o_ref):
    s = p_ref[0, :N2] + p_ref[1, :N2] + b2_ref[...]
    d = s[:N] - s[N:N2] + 1e-6
    o_ref[...] = jnp.sqrt(jnp.sum(d * d, axis=1))


def _final(parts, b2):
    return pl.pallas_call(
        _final_body,
        out_shape=jax.ShapeDtypeStruct((N,), jnp.float32),
    )(parts, b2)


def kernel(x1, adj1, x2, adj2, W1, b1, W2, b2):
    x = jnp.concatenate([x1, x2], axis=0)
    src = jnp.concatenate([adj1[0], adj2[0] + N])
    dst = jnp.concatenate([adj1[1], adj2[1] + N])
    pad = E_PAD - NE2
    src = jnp.concatenate([src, jnp.zeros((pad,), jnp.int32)])
    dst = jnp.concatenate([dst, jnp.full((pad,), SINK, jnp.int32)])
    src2d = src.reshape(ROWS_PAD, CHUNK)
    dst2d = dst.reshape(ROWS_PAD, CHUNK)
    zeros64 = jnp.zeros((NP, NHID), jnp.float32)
    zeros32 = jnp.zeros((NP, NOUT), jnp.float32)

    support = _mm1(x, W1)
    agg1 = _spmm(NHID)(support, src2d, dst2d, zeros64)
    support2 = _fused2(agg1, b1.reshape(1, NHID), W2)
    agg2 = _spmm(NOUT)(support2, src2d, dst2d, zeros32)
    return _final(agg2, b2.reshape(1, NOUT))


# speedup vs baseline: 5.4363x; 1.2029x over previous
import functools

import jax
import jax.numpy as jnp
from jax import lax
from jax.experimental import pallas as pl
from jax.experimental.pallas import tpu as pltpu
from jax.experimental.pallas import tpu_sc as plsc

N = 10000
E = 320000
NFEAT = 128
NHID = 64
NOUT = 32

N2 = 2 * N
NE2 = 2 * E
CHUNK = 128
NCORES = 2
NSUB = 16
NW = NCORES * NSUB
ROWS = NE2 // CHUNK
RPW = -(-ROWS // NW)
RPW = -(-RPW // 8) * 8
ROWS_PAD = RPW * NW
E_PAD = ROWS_PAD * CHUNK
SINK = N2
NBUF = 4
IB = 16
NIB = RPW // IB
ZPT = 1256
NP = ZPT * NSUB


def _make_spmm(width):
    mesh = plsc.VectorSubcoreMesh(
        core_axis_name="c", subcore_axis_name="s",
        num_cores=NCORES, num_subcores=NSUB)

    @functools.partial(
        pl.kernel,
        out_type=jax.ShapeDtypeStruct((NCORES, NP, width), jnp.float32),
        mesh=mesh,
        scratch_types=[
            pltpu.VMEM((IB, 2, CHUNK), jnp.int32),
            pltpu.VMEM((IB, 2, CHUNK), jnp.int32),
            pltpu.SemaphoreType.DMA,
            pltpu.SemaphoreType.DMA,
        ] + [pltpu.VMEM((CHUNK, width), jnp.float32) for _ in range(NBUF)]
          + [pltpu.SemaphoreType.DMA for _ in range(2 * NBUF)]
          + [pltpu.VMEM_SHARED((NP, width), jnp.float32)],
        compiler_params=pltpu.CompilerParams(use_tc_tiling_on_sc=False),
    )
    def spmm(table_hbm, idx_hbm, zeros_hbm, out_hbm,
             ib0, ib1, is0, is1, *bufs_and_sems):
        ibuf = (ib0, ib1)
        isem = (is0, is1)
        bufs = bufs_and_sems[:NBUF]
        gsem = bufs_and_sems[NBUF:2 * NBUF]
        ssem = bufs_and_sems[2 * NBUF:3 * NBUF]
        acc = bufs_and_sems[3 * NBUF]
        cid = lax.axis_index("c")
        sid = lax.axis_index("s")
        wid = sid * NCORES + cid
        base = wid * RPW
        pltpu.sync_copy(zeros_hbm.at[pl.ds(sid * ZPT, ZPT)],
                        acc.at[pl.ds(sid * ZPT, ZPT)])
        plsc.subcore_barrier()

        pltpu.async_copy(idx_hbm.at[pl.ds(base, IB)], ibuf[0], isem[0])
        pltpu.async_copy(idx_hbm.at[pl.ds(base + IB, IB)], ibuf[1], isem[1])
        pltpu.make_async_copy(idx_hbm.at[pl.ds(base, IB)], ibuf[0],
                              isem[0]).wait()
        for b in range(NBUF):
            pltpu.async_copy(table_hbm.at[ibuf[0].at[b, 0]], bufs[b], gsem[b])

        @pl.loop(0, NIB, step=2)
        def _(k0):
            for p in range(2):
                kk = k0 + p
                cur, nxt = ibuf[p], ibuf[1 - p]
                for j in range(IB):
                    b = j % NBUF
                    if j == IB - NBUF:
                        @pl.when(kk + 1 < NIB)
                        def _():
                            pltpu.make_async_copy(
                                idx_hbm.at[pl.ds(base, IB)], nxt,
                                isem[1 - p]).wait()
                    pltpu.make_async_copy(table_hbm.at[cur.at[j, 0]],
                                          bufs[b], gsem[b]).wait()
                    pltpu.async_copy(bufs[b], acc.at[cur.at[j, 1]], ssem[b],
                                     add=True)
                    pltpu.make_async_copy(bufs[b], acc.at[cur.at[j, 1]],
                                          ssem[b]).wait()
                    jn = j + NBUF
                    if jn < IB:
                        pltpu.async_copy(table_hbm.at[cur.at[jn, 0]],
                                         bufs[b], gsem[b])
                    else:

                        @pl.when(kk + 1 < NIB)
                        def _(b=b, jn=jn):
                            pltpu.async_copy(
                                table_hbm.at[nxt.at[jn - IB, 0]],
                                bufs[b], gsem[b])

                @pl.when(kk + 2 < NIB)
                def _():
                    pltpu.async_copy(
                        idx_hbm.at[pl.ds(base + (kk + 2) * IB, IB)],
                        cur, isem[p])

        plsc.subcore_barrier()
        pltpu.sync_copy(acc.at[pl.ds(sid * ZPT, ZPT)],
                        out_hbm.at[cid, pl.ds(sid * ZPT, ZPT)])

    return spmm


_spmm_cache = {}


def _spmm(width):
    if width not in _spmm_cache:
        _spmm_cache[width] = _make_spmm(width)
    return _spmm_cache[width]


_MMBLK = 2000


def _mm1_body(x_ref, w_ref, o_ref):
    o_ref[...] = jnp.dot(x_ref[...], w_ref[...],
                         preferred_element_type=jnp.float32)


def _mm1(x, w1):
    return pl.pallas_call(
        _mm1_body,
        grid=(N2 // _MMBLK,),
        in_specs=[
            pl.BlockSpec((_MMBLK, NFEAT), lambda i: (i, 0)),
            pl.BlockSpec((NFEAT, NHID), lambda i: (0, 0)),
        ],
        out_specs=pl.BlockSpec((_MMBLK, NHID), lambda i: (i, 0)),
        out_shape=jax.ShapeDtypeStruct((N2, NHID), jnp.float32),
    )(x, w1)


def _fused2_body(p0_ref, p1_ref, b1_ref, w2_ref, o_ref):
    a = p0_ref[0] + p1_ref[0] + b1_ref[...]
    h = jnp.maximum(a, 0.0)
    o_ref[...] = jnp.dot(h, w2_ref[...], preferred_element_type=jnp.float32)


def _fused2(parts, b1, w2):
    return pl.pallas_call(
        _fused2_body,
        grid=(N2 // _MMBLK,),
        in_specs=[
            pl.BlockSpec((1, _MMBLK, NHID), lambda i: (0, i, 0)),
            pl.BlockSpec((1, _MMBLK, NHID), lambda i: (1, i, 0)),
            pl.BlockSpec((1, NHID), lambda i: (0, 0)),
            pl.BlockSpec((NHID, NOUT), lambda i: (0, 0)),
        ],
        out_specs=pl.BlockSpec((_MMBLK, NOUT), lambda i: (i, 0)),
        out_shape=jax.ShapeDtypeStruct((N2, NOUT), jnp.float32),
    )(parts, parts, b1, w2)


def _final_body(p_ref, b2_ref, o_ref):
    s = p_ref[0, :N2] + p_ref[1, :N2] + b2_ref[...]
    d = s[:N] - s[N:N2] + 1e-6
    o_ref[...] = jnp.sqrt(jnp.sum(d * d, axis=1))


def _final(parts, b2):
    return pl.pallas_call(
        _final_body,
        out_shape=jax.ShapeDtypeStruct((N,), jnp.float32),
    )(parts, b2)


def kernel(x1, adj1, x2, adj2, W1, b1, W2, b2):
    x = jnp.concatenate([x1, x2], axis=0)
    src = jnp.concatenate([adj1[0], adj2[0] + N])
    dst = jnp.concatenate([adj1[1], adj2[1] + N])
    pad = E_PAD - NE2
    src = jnp.concatenate([src, jnp.zeros((pad,), jnp.int32)])
    dst = jnp.concatenate([dst, jnp.full((pad,), SINK, jnp.int32)])
    idx2 = jnp.stack([src.reshape(ROWS_PAD, CHUNK),
                      dst.reshape(ROWS_PAD, CHUNK)], axis=1)
    zeros64 = jnp.zeros((NP, NHID), jnp.float32)
    zeros32 = jnp.zeros((NP, NOUT), jnp.float32)

    support = _mm1(x, W1)
    agg1 = _spmm(NHID)(support, idx2, zeros64)
    support2 = _fused2(agg1, b1.reshape(1, NHID), W2)
    agg2 = _spmm(NOUT)(support2, idx2, zeros32)
    return _final(agg2, b2.reshape(1, NOUT))


# speedup vs baseline: 15.4629x; 2.8444x over previous
import functools

import jax
import jax.numpy as jnp
from jax import lax
from jax.experimental import pallas as pl
from jax.experimental.pallas import tpu as pltpu
from jax.experimental.pallas import tpu_sc as plsc

N = 10000
E = 320000
NFEAT = 128
NHID = 64
NOUT = 32

N2 = 2 * N
NE2 = 2 * E
CHUNK = 128
NCORES = 2
NSUB = 16
NW = NCORES * NSUB
ROWS = NE2 // CHUNK
RPW = -(-ROWS // NW)
RPW = -(-RPW // 8) * 8
ROWS_PAD = RPW * NW
E_PAD = ROWS_PAD * CHUNK
SINK = N2
NBUF = 4
IB = 16
NIB = RPW // IB
ZPT = 1256
NP = ZPT * NSUB


def _make_spmm(width):
    mesh = plsc.VectorSubcoreMesh(
        core_axis_name="c", subcore_axis_name="s",
        num_cores=NCORES, num_subcores=NSUB)

    @functools.partial(
        pl.kernel,
        out_type=jax.ShapeDtypeStruct((NCORES, NP, width), jnp.float32),
        mesh=mesh,
        scratch_types=[
            pltpu.VMEM((IB, 2, CHUNK), jnp.int32),
            pltpu.VMEM((IB, 2, CHUNK), jnp.int32),
            pltpu.SemaphoreType.DMA,
            pltpu.SemaphoreType.DMA,
        ] + [pltpu.VMEM((CHUNK, width), jnp.float32) for _ in range(NBUF)]
          + [pltpu.SemaphoreType.DMA for _ in range(2 * NBUF)]
          + [pltpu.VMEM_SHARED((NP, width), jnp.float32)],
        compiler_params=pltpu.CompilerParams(use_tc_tiling_on_sc=False),
    )
    def spmm(table_hbm, idx_hbm, zeros_hbm, out_hbm,
             ib0, ib1, is0, is1, *bufs_and_sems):
        ibuf = (ib0, ib1)
        isem = (is0, is1)
        bufs = bufs_and_sems[:NBUF]
        gsem = bufs_and_sems[NBUF:2 * NBUF]
        ssem = bufs_and_sems[2 * NBUF:3 * NBUF]
        acc = bufs_and_sems[3 * NBUF]
        cid = lax.axis_index("c")
        sid = lax.axis_index("s")
        wid = sid * NCORES + cid
        base = wid * RPW
        pltpu.sync_copy(zeros_hbm.at[pl.ds(sid * ZPT, ZPT)],
                        acc.at[pl.ds(sid * ZPT, ZPT)])
        plsc.subcore_barrier()

        pltpu.async_copy(idx_hbm.at[pl.ds(base, IB)], ibuf[0], isem[0])
        pltpu.async_copy(idx_hbm.at[pl.ds(base + IB, IB)], ibuf[1], isem[1])
        pltpu.make_async_copy(idx_hbm.at[pl.ds(base, IB)], ibuf[0],
                              isem[0]).wait()
        for b in range(NBUF):
            pltpu.async_copy(table_hbm.at[ibuf[0].at[b, 0]], bufs[b], gsem[b])

        @pl.loop(0, NIB, step=2)
        def _(k0):
            for p in range(2):
                kk = k0 + p
                cur, nxt = ibuf[p], ibuf[1 - p]
                for j in range(IB):
                    b = j % NBUF
                    if j == IB - NBUF:
                        @pl.when(kk + 1 < NIB)
                        def _():
                            pltpu.make_async_copy(
                                idx_hbm.at[pl.ds(base, IB)], nxt,
                                isem[1 - p]).wait()
                    pltpu.make_async_copy(table_hbm.at[cur.at[j, 0]],
                                          bufs[b], gsem[b]).wait()
                    pltpu.async_copy(bufs[b], acc.at[cur.at[j, 1]], ssem[b],
                                     add=True)
                    pltpu.make_async_copy(bufs[b], acc.at[cur.at[j, 1]],
                                          ssem[b]).wait()
                    jn = j + NBUF
                    if jn < IB:
                        pltpu.async_copy(table_hbm.at[cur.at[jn, 0]],
                                         bufs[b], gsem[b])
                    else:

                        @pl.when(kk + 1 < NIB)
                        def _(b=b, jn=jn):
                            pltpu.async_copy(
                                table_hbm.at[nxt.at[jn - IB, 0]],
                                bufs[b], gsem[b])

                @pl.when(kk + 2 < NIB)
                def _():
                    pltpu.async_copy(
                        idx_hbm.at[pl.ds(base + (kk + 2) * IB, IB)],
                        cur, isem[p])

        plsc.subcore_barrier()
        pltpu.sync_copy(acc.at[pl.ds(sid * ZPT, ZPT)],
                        out_hbm.at[cid, pl.ds(sid * ZPT, ZPT)])

    return spmm


_spmm_cache = {}


def _spmm(width):
    if width not in _spmm_cache:
        _spmm_cache[width] = _make_spmm(width)
    return _spmm_cache[width]


_MMBLK = 2000


def _mm1_body(x_ref, w_ref, o_ref):
    o_ref[...] = jnp.dot(x_ref[...], w_ref[...],
                         preferred_element_type=jnp.float32)


def _mm1(x, w1):
    return pl.pallas_call(
        _mm1_body,
        grid=(N2 // _MMBLK,),
        in_specs=[
            pl.BlockSpec((_MMBLK, NFEAT), lambda i: (i, 0)),
            pl.BlockSpec((NFEAT, NHID), lambda i: (0, 0)),
        ],
        out_specs=pl.BlockSpec((_MMBLK, NHID), lambda i: (i, 0)),
        out_shape=jax.ShapeDtypeStruct((N2, NHID), jnp.float32),
    )(x, w1)


def _fused2_body(p0_ref, p1_ref, b1_ref, w2_ref, o_ref):
    a = p0_ref[0] + p1_ref[0] + b1_ref[...]
    h = jnp.maximum(a, 0.0)
    o_ref[...] = jnp.dot(h, w2_ref[...], preferred_element_type=jnp.float32)


def _fused2(parts, b1, w2):
    return pl.pallas_call(
        _fused2_body,
        grid=(N2 // _MMBLK,),
        in_specs=[
            pl.BlockSpec((1, _MMBLK, NHID), lambda i: (0, i, 0)),
            pl.BlockSpec((1, _MMBLK, NHID), lambda i: (1, i, 0)),
            pl.BlockSpec((1, NHID), lambda i: (0, 0)),
            pl.BlockSpec((NHID, NOUT), lambda i: (0, 0)),
        ],
        out_specs=pl.BlockSpec((_MMBLK, NOUT), lambda i: (i, 0)),
        out_shape=jax.ShapeDtypeStruct((N2, NOUT), jnp.float32),
    )(parts, parts, b1, w2)


def _final_body(p_ref, b2_ref, o_ref):
    s = p_ref[0, :N2] + p_ref[1, :N2] + b2_ref[...]
    d = s[:N] - s[N:N2] + 1e-6
    o_ref[...] = jnp.sqrt(jnp.sum(d * d, axis=1))


def _final(parts, b2):
    return pl.pallas_call(
        _final_body,
        out_shape=jax.ShapeDtypeStruct((N,), jnp.float32),
    )(parts, b2)


def kernel(x1, adj1, x2, adj2, W1, b1, W2, b2):
    x = jnp.concatenate([x1, x2], axis=0)
    src = jnp.concatenate([adj1[0], adj2[0] + N])
    dst = jnp.concatenate([adj1[1], adj2[1] + N])
    pad = E_PAD - NE2
    pad_iota = jnp.arange(pad, dtype=jnp.int32)
    src = jnp.concatenate([src, pad_iota % N2])
    dst = jnp.concatenate([dst, SINK + pad_iota % (NP - N2)])
    idx2 = jnp.stack([src.reshape(ROWS_PAD, CHUNK),
                      dst.reshape(ROWS_PAD, CHUNK)], axis=1)
    zeros64 = jnp.zeros((NP, NHID), jnp.float32)
    zeros32 = jnp.zeros((NP, NOUT), jnp.float32)

    support = _mm1(x, W1)
    agg1 = _spmm(NHID)(support, idx2, zeros64)
    support2 = _fused2(agg1, b1.reshape(1, NHID), W2)
    agg2 = _spmm(NOUT)(support2, idx2, zeros32)
    return _final(agg2, b2.reshape(1, NOUT))


# speedup vs baseline: 18.2912x; 1.1829x over previous
import functools

import jax
import jax.numpy as jnp
from jax import lax
from jax.experimental import pallas as pl
from jax.experimental.pallas import tpu as pltpu
from jax.experimental.pallas import tpu_sc as plsc

N = 10000
E = 320000
NFEAT = 128
NHID = 64
NOUT = 32

N2 = 2 * N
CHUNK = 125
NCORES = 2
NSUB = 16
CPG = E // CHUNK
RPW = CPG // NSUB
NBUF = 4
IB = 16
NIB = RPW // IB
ZPT = N // NSUB


def _make_spmm(width):
    mesh = plsc.VectorSubcoreMesh(
        core_axis_name="c", subcore_axis_name="s",
        num_cores=NCORES, num_subcores=NSUB)

    @functools.partial(
        pl.kernel,
        out_type=jax.ShapeDtypeStruct((NCORES, N, width), jnp.float32),
        mesh=mesh,
        scratch_types=[
            pltpu.VMEM((2, IB, CHUNK), jnp.int32),
            pltpu.VMEM((2, IB, CHUNK), jnp.int32),
            pltpu.SemaphoreType.DMA,
            pltpu.SemaphoreType.DMA,
        ] + [pltpu.VMEM((CHUNK, width), jnp.float32) for _ in range(NBUF)]
          + [pltpu.SemaphoreType.DMA for _ in range(2 * NBUF)]
          + [pltpu.VMEM_SHARED((N, width), jnp.float32)],
        compiler_params=pltpu.CompilerParams(use_tc_tiling_on_sc=False),
    )
    def spmm(table_hbm, adj1_hbm, adj2_hbm, zeros_hbm, out_hbm,
             ib0, ib1, is0, is1, *bufs_and_sems):
        ibuf = (ib0, ib1)
        isem = (is0, is1)
        bufs = bufs_and_sems[:NBUF]
        gsem = bufs_and_sems[NBUF:2 * NBUF]
        ssem = bufs_and_sems[2 * NBUF:3 * NBUF]
        acc = bufs_and_sems[3 * NBUF]
        cid = lax.axis_index("c")
        sid = lax.axis_index("s")
        base = sid * RPW
        pltpu.sync_copy(zeros_hbm.at[pl.ds(sid * ZPT, ZPT)],
                        acc.at[pl.ds(sid * ZPT, ZPT)])
        plsc.subcore_barrier()

        def load_idx(block, dst, sem):
            @pl.when(cid == 0)
            def _():
                pltpu.async_copy(adj1_hbm.at[0, pl.ds(block * IB, IB)],
                                 dst.at[0], sem)
                pltpu.async_copy(adj1_hbm.at[1, pl.ds(block * IB, IB)],
                                 dst.at[1], sem)

            @pl.when(cid == 1)
            def _():
                pltpu.async_copy(adj2_hbm.at[0, pl.ds(block * IB, IB)],
                                 dst.at[0], sem)
                pltpu.async_copy(adj2_hbm.at[1, pl.ds(block * IB, IB)],
                                 dst.at[1], sem)

        def wait_idx(dst, sem):
            pltpu.make_async_copy(adj1_hbm.at[0, pl.ds(0, IB)],
                                  dst.at[0], sem).wait()
            pltpu.make_async_copy(adj1_hbm.at[1, pl.ds(0, IB)],
                                  dst.at[1], sem).wait()

        table = table_hbm.at[cid]

        b0 = base // IB
        load_idx(b0, ibuf[0], isem[0])
        load_idx(b0 + 1, ibuf[1], isem[1])
        wait_idx(ibuf[0], isem[0])
        for b in range(NBUF):
            pltpu.async_copy(table.at[ibuf[0].at[0, b]], bufs[b], gsem[b])

        @pl.loop(0, NIB, step=2)
        def _(k0):
            for p in range(2):
                kk = k0 + p
                cur, nxt = ibuf[p], ibuf[1 - p]
                for j in range(IB):
                    b = j % NBUF
                    if j == IB - NBUF:
                        @pl.when(kk + 1 < NIB)
                        def _():
                            wait_idx(nxt, isem[1 - p])
                    pltpu.make_async_copy(table.at[cur.at[0, j]],
                                          bufs[b], gsem[b]).wait()
                    pltpu.async_copy(bufs[b], acc.at[cur.at[1, j]], ssem[b],
                                     add=True)
                    pltpu.make_async_copy(bufs[b], acc.at[cur.at[1, j]],
                                          ssem[b]).wait()
                    jn = j + NBUF
                    if jn < IB:
                        pltpu.async_copy(table.at[cur.at[0, jn]],
                                         bufs[b], gsem[b])
                    else:

                        @pl.when(kk + 1 < NIB)
                        def _(b=b, jn=jn):
                            pltpu.async_copy(table.at[nxt.at[0, jn - IB]],
                                             bufs[b], gsem[b])

                @pl.when(kk + 2 < NIB)
                def _():
                    load_idx(b0 + kk + 2, cur, isem[p])

        plsc.subcore_barrier()
        pltpu.sync_copy(acc.at[pl.ds(sid * ZPT, ZPT)],
                        out_hbm.at[cid, pl.ds(sid * ZPT, ZPT)])

    return spmm


_spmm_cache = {}


def _spmm(width):
    if width not in _spmm_cache:
        _spmm_cache[width] = _make_spmm(width)
    return _spmm_cache[width]


_MMBLK = 2000


def _mm1_body(x_ref, w_ref, o_ref):
    o_ref[...] = jnp.dot(x_ref[...], w_ref[...],
                         preferred_element_type=jnp.float32)


def _mm1(x, w1):
    return pl.pallas_call(
        _mm1_body,
        grid=(N2 // _MMBLK,),
        in_specs=[
            pl.BlockSpec((_MMBLK, NFEAT), lambda i: (i, 0)),
            pl.BlockSpec((NFEAT, NHID), lambda i: (0, 0)),
        ],
        out_specs=pl.BlockSpec((_MMBLK, NHID), lambda i: (i, 0)),
        out_shape=jax.ShapeDtypeStruct((N2, NHID), jnp.float32),
    )(x, w1)


def _fused2_body(p_ref, b1_ref, w2_ref, o_ref):
    a = p_ref[0] + b1_ref[...]
    h = jnp.maximum(a, 0.0)
    o_ref[0] = jnp.dot(h, w2_ref[...], preferred_element_type=jnp.float32)


def _fused2(parts, b1, w2):
    return pl.pallas_call(
        _fused2_body,
        grid=(NCORES, N // _MMBLK),
        in_specs=[
            pl.BlockSpec((1, _MMBLK, NHID), lambda g, i: (g, i, 0)),
            pl.BlockSpec((1, NHID), lambda g, i: (0, 0)),
            pl.BlockSpec((NHID, NOUT), lambda g, i: (0, 0)),
        ],
        out_specs=pl.BlockSpec((1, _MMBLK, NOUT), lambda g, i: (g, i, 0)),
        out_shape=jax.ShapeDtypeStruct((NCORES, N, NOUT), jnp.float32),
    )(parts, b1, w2)


def _final_body(p_ref, o_ref):
    d = p_ref[0] - p_ref[1] + 1e-6
    o_ref[...] = jnp.sqrt(jnp.sum(d * d, axis=1))


def _final(parts):
    return pl.pallas_call(
        _final_body,
        out_shape=jax.ShapeDtypeStruct((N,), jnp.float32),
    )(parts)


def kernel(x1, adj1, x2, adj2, W1, b1, W2, b2):
    x = jnp.concatenate([x1, x2], axis=0)
    adj1r = adj1.reshape(2, CPG, CHUNK)
    adj2r = adj2.reshape(2, CPG, CHUNK)
    zeros64 = jnp.zeros((N, NHID), jnp.float32)
    zeros32 = jnp.zeros((N, NOUT), jnp.float32)

    support = _mm1(x, W1).reshape(NCORES, N, NHID)
    agg1 = _spmm(NHID)(support, adj1r, adj2r, zeros64)
    support2 = _fused2(agg1, b1.reshape(1, NHID), W2)
    agg2 = _spmm(NOUT)(support2, adj1r, adj2r, zeros32)
    return _final(agg2)


# speedup vs baseline: 18.9457x; 1.0358x over previous
import functools

import jax
import jax.numpy as jnp
from jax import lax
from jax.experimental import pallas as pl
from jax.experimental.pallas import tpu as pltpu
from jax.experimental.pallas import tpu_sc as plsc

N = 10000
E = 320000
NFEAT = 128
NHID = 64
NOUT = 32

N2 = 2 * N
CHUNK = 125
NCORES = 2
NSUB = 16
CPG = E // CHUNK
RPW = CPG // NSUB
NBUF = 6
IB = 16
NIB = RPW // IB
ZPT = N // NSUB


def _make_spmm(width):
    mesh = plsc.VectorSubcoreMesh(
        core_axis_name="c", subcore_axis_name="s",
        num_cores=NCORES, num_subcores=NSUB)

    @functools.partial(
        pl.kernel,
        out_type=jax.ShapeDtypeStruct((NCORES, N, width), jnp.float32),
        mesh=mesh,
        scratch_types=[
            pltpu.VMEM((2, IB, CHUNK), jnp.int32),
            pltpu.VMEM((2, IB, CHUNK), jnp.int32),
            pltpu.SemaphoreType.DMA,
            pltpu.SemaphoreType.DMA,
        ] + [pltpu.VMEM((CHUNK, width), jnp.float32) for _ in range(NBUF)]
          + [pltpu.SemaphoreType.DMA for _ in range(2 * NBUF)]
          + [pltpu.VMEM_SHARED((N, width), jnp.float32)],
        compiler_params=pltpu.CompilerParams(use_tc_tiling_on_sc=False),
    )
    def spmm(table_hbm, adj1_hbm, adj2_hbm, out_hbm,
             ib0, ib1, is0, is1, *bufs_and_sems):
        ibuf = (ib0, ib1)
        isem = (is0, is1)
        bufs = bufs_and_sems[:NBUF]
        gsem = bufs_and_sems[NBUF:2 * NBUF]
        ssem = bufs_and_sems[2 * NBUF:3 * NBUF]
        acc = bufs_and_sems[3 * NBUF]
        cid = lax.axis_index("c")
        sid = lax.axis_index("s")
        base = sid * RPW
        zvec = jnp.zeros((16,), jnp.float32)

        @pl.loop(0, CHUNK)
        def _(i):
            for l in range(width // 16):
                bufs[0][i, pl.ds(l * 16, 16)] = zvec

        for t in range(ZPT // CHUNK):
            pltpu.sync_copy(bufs[0],
                            acc.at[pl.ds(sid * ZPT + t * CHUNK, CHUNK)])
        plsc.subcore_barrier()

        def load_idx(block, dst, sem):
            @pl.when(cid == 0)
            def _():
                pltpu.async_copy(adj1_hbm.at[0, pl.ds(block * IB, IB)],
                                 dst.at[0], sem)
                pltpu.async_copy(adj1_hbm.at[1, pl.ds(block * IB, IB)],
                                 dst.at[1], sem)

            @pl.when(cid == 1)
            def _():
                pltpu.async_copy(adj2_hbm.at[0, pl.ds(block * IB, IB)],
                                 dst.at[0], sem)
                pltpu.async_copy(adj2_hbm.at[1, pl.ds(block * IB, IB)],
                                 dst.at[1], sem)

        def wait_idx(dst, sem):
            pltpu.make_async_copy(adj1_hbm.at[0, pl.ds(0, IB)],
                                  dst.at[0], sem).wait()
            pltpu.make_async_copy(adj1_hbm.at[1, pl.ds(0, IB)],
                                  dst.at[1], sem).wait()

        table = table_hbm.at[cid]

        b0 = base // IB
        load_idx(b0, ibuf[0], isem[0])
        load_idx(b0 + 1, ibuf[1], isem[1])
        wait_idx(ibuf[0], isem[0])
        for b in range(NBUF):
            pltpu.async_copy(table.at[ibuf[0].at[0, b]], bufs[b], gsem[b])

        @pl.loop(0, NIB, step=2)
        def _(k0):
            for p in range(2):
                kk = k0 + p
                cur, nxt = ibuf[p], ibuf[1 - p]
                for j in range(IB):
                    b = j % NBUF
                    if j == IB - NBUF:
                        @pl.when(kk + 1 < NIB)
                        def _():
                            wait_idx(nxt, isem[1 - p])
                    pltpu.make_async_copy(table.at[cur.at[0, j]],
                                          bufs[b], gsem[b]).wait()
                    pltpu.async_copy(bufs[b], acc.at[cur.at[1, j]], ssem[b],
                                     add=True)
                    pltpu.make_async_copy(bufs[b], acc.at[cur.at[1, j]],
                                          ssem[b]).wait()
                    jn = j + NBUF
                    if jn < IB:
                        pltpu.async_copy(table.at[cur.at[0, jn]],
                                         bufs[b], gsem[b])
                    else:

                        @pl.when(kk + 1 < NIB)
                        def _(b=b, jn=jn):
                            pltpu.async_copy(table.at[nxt.at[0, jn - IB]],
                                             bufs[b], gsem[b])

                @pl.when(kk + 2 < NIB)
                def _():
                    load_idx(b0 + kk + 2, cur, isem[p])

        plsc.subcore_barrier()
        pltpu.sync_copy(acc.at[pl.ds(sid * ZPT, ZPT)],
                        out_hbm.at[cid, pl.ds(sid * ZPT, ZPT)])

    return spmm


_spmm_cache = {}


def _spmm(width):
    if width not in _spmm_cache:
        _spmm_cache[width] = _make_spmm(width)
    return _spmm_cache[width]


_MMBLK = 2000


def _mm1_body(x_ref, w_ref, o_ref):
    o_ref[...] = jnp.dot(x_ref[...], w_ref[...],
                         preferred_element_type=jnp.float32)


def _mm1(x, w1):
    return pl.pallas_call(
        _mm1_body,
        grid=(N2 // _MMBLK,),
        in_specs=[
            pl.BlockSpec((_MMBLK, NFEAT), lambda i: (i, 0)),
            pl.BlockSpec((NFEAT, NHID), lambda i: (0, 0)),
        ],
        out_specs=pl.BlockSpec((_MMBLK, NHID), lambda i: (i, 0)),
        out_shape=jax.ShapeDtypeStruct((N2, NHID), jnp.float32),
    )(x, w1)


def _fused2_body(p_ref, b1_ref, w2_ref, o_ref):
    a = p_ref[0] + b1_ref[...]
    h = jnp.maximum(a, 0.0)
    o_ref[0] = jnp.dot(h, w2_ref[...], preferred_element_type=jnp.float32)


def _fused2(parts, b1, w2):
    return pl.pallas_call(
        _fused2_body,
        grid=(NCORES, N // _MMBLK),
        in_specs=[
            pl.BlockSpec((1, _MMBLK, NHID), lambda g, i: (g, i, 0)),
            pl.BlockSpec((1, NHID), lambda g, i: (0, 0)),
            pl.BlockSpec((NHID, NOUT), lambda g, i: (0, 0)),
        ],
        out_specs=pl.BlockSpec((1, _MMBLK, NOUT), lambda g, i: (g, i, 0)),
        out_shape=jax.ShapeDtypeStruct((NCORES, N, NOUT), jnp.float32),
    )(parts, b1, w2)


def _final_body(p_ref, o_ref):
    d = p_ref[0] - p_ref[1] + 1e-6
    o_ref[...] = jnp.sqrt(jnp.sum(d * d, axis=1))


def _final(parts):
    return pl.pallas_call(
        _final_body,
        out_shape=jax.ShapeDtypeStruct((N,), jnp.float32),
    )(parts)


def kernel(x1, adj1, x2, adj2, W1, b1, W2, b2):
    x = jnp.concatenate([x1, x2], axis=0)
    adj1r = adj1.reshape(2, CPG, CHUNK)
    adj2r = adj2.reshape(2, CPG, CHUNK)

    support = _mm1(x, W1).reshape(NCORES, N, NHID)
    agg1 = _spmm(NHID)(support, adj1r, adj2r)
    support2 = _fused2(agg1, b1.reshape(1, NHID), W2)
    agg2 = _spmm(NOUT)(support2, adj1r, adj2r)
    return _final(agg2)


# speedup vs baseline: 19.1469x; 1.0106x over previous
import functools

import jax
import jax.numpy as jnp
from jax import lax
from jax.experimental import pallas as pl
from jax.experimental.pallas import tpu as pltpu
from jax.experimental.pallas import tpu_sc as plsc

N = 10000
E = 320000
NFEAT = 128
NHID = 64
NOUT = 32

N2 = 2 * N
CHUNK = 125
NCORES = 2
NSUB = 16
CPG = E // CHUNK
RPW = CPG // NSUB
NBUF = 8
IB = 16
NIB = RPW // IB
ZPT = N // NSUB


def _make_spmm(width):
    mesh = plsc.VectorSubcoreMesh(
        core_axis_name="c", subcore_axis_name="s",
        num_cores=NCORES, num_subcores=NSUB)

    @functools.partial(
        pl.kernel,
        out_type=jax.ShapeDtypeStruct((NCORES, N, width), jnp.float32),
        mesh=mesh,
        scratch_types=[
            pltpu.VMEM((2, IB, CHUNK), jnp.int32),
            pltpu.VMEM((2, IB, CHUNK), jnp.int32),
            pltpu.SemaphoreType.DMA,
            pltpu.SemaphoreType.DMA,
        ] + [pltpu.VMEM((CHUNK, width), jnp.float32) for _ in range(NBUF)]
          + [pltpu.SemaphoreType.DMA for _ in range(2 * NBUF)]
          + [pltpu.VMEM_SHARED((N, width), jnp.float32)],
        compiler_params=pltpu.CompilerParams(use_tc_tiling_on_sc=False),
    )
    def spmm(table_hbm, adj1_hbm, adj2_hbm, out_hbm,
             ib0, ib1, is0, is1, *bufs_and_sems):
        ibuf = (ib0, ib1)
        isem = (is0, is1)
        bufs = bufs_and_sems[:NBUF]
        gsem = bufs_and_sems[NBUF:2 * NBUF]
        ssem = bufs_and_sems[2 * NBUF:3 * NBUF]
        acc = bufs_and_sems[3 * NBUF]
        cid = lax.axis_index("c")
        sid = lax.axis_index("s")
        base = sid * RPW
        zvec = jnp.zeros((16,), jnp.float32)

        @pl.loop(0, CHUNK)
        def _(i):
            for l in range(width // 16):
                bufs[0][i, pl.ds(l * 16, 16)] = zvec

        for t in range(ZPT // CHUNK):
            pltpu.sync_copy(bufs[0],
                            acc.at[pl.ds(sid * ZPT + t * CHUNK, CHUNK)])
        plsc.subcore_barrier()

        def load_idx(block, dst, sem):
            @pl.when(cid == 0)
            def _():
                pltpu.async_copy(adj1_hbm.at[0, pl.ds(block * IB, IB)],
                                 dst.at[0], sem)
                pltpu.async_copy(adj1_hbm.at[1, pl.ds(block * IB, IB)],
                                 dst.at[1], sem)

            @pl.when(cid == 1)
            def _():
                pltpu.async_copy(adj2_hbm.at[0, pl.ds(block * IB, IB)],
                                 dst.at[0], sem)
                pltpu.async_copy(adj2_hbm.at[1, pl.ds(block * IB, IB)],
                                 dst.at[1], sem)

        def wait_idx(dst, sem):
            pltpu.make_async_copy(adj1_hbm.at[0, pl.ds(0, IB)],
                                  dst.at[0], sem).wait()
            pltpu.make_async_copy(adj1_hbm.at[1, pl.ds(0, IB)],
                                  dst.at[1], sem).wait()

        table = table_hbm.at[cid]

        b0 = base // IB
        load_idx(b0, ibuf[0], isem[0])
        load_idx(b0 + 1, ibuf[1], isem[1])
        wait_idx(ibuf[0], isem[0])
        for b in range(NBUF):
            pltpu.async_copy(table.at[ibuf[0].at[0, b]], bufs[b], gsem[b])

        @pl.loop(0, NIB, step=2)
        def _(k0):
            for p in range(2):
                kk = k0 + p
                cur, nxt = ibuf[p], ibuf[1 - p]
                for j in range(IB):
                    b = j % NBUF
                    if j == IB - NBUF:
                        @pl.when(kk + 1 < NIB)
                        def _():
                            wait_idx(nxt, isem[1 - p])
                    pltpu.make_async_copy(table.at[cur.at[0, j]],
                                          bufs[b], gsem[b]).wait()
                    pltpu.async_copy(bufs[b], acc.at[cur.at[1, j]], ssem[b],
                                     add=True)
                    pltpu.make_async_copy(bufs[b], acc.at[cur.at[1, j]],
                                          ssem[b]).wait()
                    jn = j + NBUF
                    if jn < IB:
                        pltpu.async_copy(table.at[cur.at[0, jn]],
                                         bufs[b], gsem[b])
                    else:

                        @pl.when(kk + 1 < NIB)
                        def _(b=b, jn=jn):
                            pltpu.async_copy(table.at[nxt.at[0, jn - IB]],
                                             bufs[b], gsem[b])

                @pl.when(kk + 2 < NIB)
                def _():
                    load_idx(b0 + kk + 2, cur, isem[p])

        plsc.subcore_barrier()
        pltpu.sync_copy(acc.at[pl.ds(sid * ZPT, ZPT)],
                        out_hbm.at[cid, pl.ds(sid * ZPT, ZPT)])

    return spmm


_spmm_cache = {}


def _spmm(width):
    if width not in _spmm_cache:
        _spmm_cache[width] = _make_spmm(width)
    return _spmm_cache[width]


_MMBLK = 2000


def _mm1_body(x_ref, w_ref, o_ref):
    o_ref[...] = jnp.dot(x_ref[...], w_ref[...],
                         preferred_element_type=jnp.float32)


def _mm1(x, w1):
    return pl.pallas_call(
        _mm1_body,
        grid=(N2 // _MMBLK,),
        in_specs=[
            pl.BlockSpec((_MMBLK, NFEAT), lambda i: (i, 0)),
            pl.BlockSpec((NFEAT, NHID), lambda i: (0, 0)),
        ],
        out_specs=pl.BlockSpec((_MMBLK, NHID), lambda i: (i, 0)),
        out_shape=jax.ShapeDtypeStruct((N2, NHID), jnp.float32),
    )(x, w1)


def _fused2_body(p_ref, b1_ref, w2_ref, o_ref):
    a = p_ref[0] + b1_ref[...]
    h = jnp.maximum(a, 0.0)
    o_ref[0] = jnp.dot(h, w2_ref[...], preferred_element_type=jnp.float32)


def _fused2(parts, b1, w2):
    return pl.pallas_call(
        _fused2_body,
        grid=(NCORES, N // _MMBLK),
        in_specs=[
            pl.BlockSpec((1, _MMBLK, NHID), lambda g, i: (g, i, 0)),
            pl.BlockSpec((1, NHID), lambda g, i: (0, 0)),
            pl.BlockSpec((NHID, NOUT), lambda g, i: (0, 0)),
        ],
        out_specs=pl.BlockSpec((1, _MMBLK, NOUT), lambda g, i: (g, i, 0)),
        out_shape=jax.ShapeDtypeStruct((NCORES, N, NOUT), jnp.float32),
    )(parts, b1, w2)


def _final_body(p_ref, o_ref):
    d = p_ref[0] - p_ref[1] + 1e-6
    o_ref[...] = jnp.sqrt(jnp.sum(d * d, axis=1))


def _final(parts):
    return pl.pallas_call(
        _final_body,
        out_shape=jax.ShapeDtypeStruct((N,), jnp.float32),
    )(parts)


def kernel(x1, adj1, x2, adj2, W1, b1, W2, b2):
    x = jnp.concatenate([x1, x2], axis=0)
    adj1r = adj1.reshape(2, CPG, CHUNK)
    adj2r = adj2.reshape(2, CPG, CHUNK)

    support = _mm1(x, W1).reshape(NCORES, N, NHID)
    agg1 = _spmm(NHID)(support, adj1r, adj2r)
    support2 = _fused2(agg1, b1.reshape(1, NHID), W2)
    agg2 = _spmm(NOUT)(support2, adj1r, adj2r)
    return _final(agg2)


# speedup vs baseline: 19.7326x; 1.0306x over previous
import functools

import jax
import jax.numpy as jnp
from jax import lax
from jax.experimental import pallas as pl
from jax.experimental.pallas import tpu as pltpu
from jax.experimental.pallas import tpu_sc as plsc

N = 10000
E = 320000
NFEAT = 128
NHID = 64
NOUT = 32

N2 = 2 * N
CHUNK = 125
NCORES = 2
NSUB = 16
CPG = E // CHUNK
RPW = CPG // NSUB
NBUF = 8
IB = 16
NIB = RPW // IB
ZPT = N // NSUB


def _make_spmm(width):
    mesh = plsc.VectorSubcoreMesh(
        core_axis_name="c", subcore_axis_name="s",
        num_cores=NCORES, num_subcores=NSUB)

    @functools.partial(
        pl.kernel,
        out_type=jax.ShapeDtypeStruct((NCORES, N, width), jnp.float32),
        mesh=mesh,
        scratch_types=[
            pltpu.VMEM((2, IB, CHUNK), jnp.int32),
            pltpu.VMEM((2, IB, CHUNK), jnp.int32),
            pltpu.SemaphoreType.DMA,
            pltpu.SemaphoreType.DMA,
        ] + [pltpu.VMEM((CHUNK, width), jnp.float32) for _ in range(NBUF)]
          + [pltpu.SemaphoreType.DMA for _ in range(2 * NBUF)]
          + [pltpu.VMEM_SHARED((N, width), jnp.float32)],
        compiler_params=pltpu.CompilerParams(use_tc_tiling_on_sc=False),
    )
    def spmm(table_hbm, adj1_hbm, adj2_hbm, out_hbm,
             ib0, ib1, is0, is1, *bufs_and_sems):
        ibuf = (ib0, ib1)
        isem = (is0, is1)
        bufs = bufs_and_sems[:NBUF]
        gsem = bufs_and_sems[NBUF:2 * NBUF]
        ssem = bufs_and_sems[2 * NBUF:3 * NBUF]
        acc = bufs_and_sems[3 * NBUF]
        cid = lax.axis_index("c")
        sid = lax.axis_index("s")
        base = sid * RPW
        zvec = jnp.zeros((16,), jnp.float32)

        @pl.loop(0, CHUNK)
        def _(i):
            for l in range(width // 16):
                bufs[0][i, pl.ds(l * 16, 16)] = zvec

        for t in range(ZPT // CHUNK):
            pltpu.sync_copy(bufs[0],
                            acc.at[pl.ds(sid * ZPT + t * CHUNK, CHUNK)])
        plsc.subcore_barrier()

        def load_idx(block, dst, sem):
            @pl.when(cid == 0)
            def _():
                pltpu.async_copy(adj1_hbm.at[0, pl.ds(block * IB, IB)],
                                 dst.at[0], sem)
                pltpu.async_copy(adj1_hbm.at[1, pl.ds(block * IB, IB)],
                                 dst.at[1], sem)

            @pl.when(cid == 1)
            def _():
                pltpu.async_copy(adj2_hbm.at[0, pl.ds(block * IB, IB)],
                                 dst.at[0], sem)
                pltpu.async_copy(adj2_hbm.at[1, pl.ds(block * IB, IB)],
                                 dst.at[1], sem)

        def wait_idx(dst, sem):
            pltpu.make_async_copy(adj1_hbm.at[0, pl.ds(0, IB)],
                                  dst.at[0], sem).wait()
            pltpu.make_async_copy(adj1_hbm.at[1, pl.ds(0, IB)],
                                  dst.at[1], sem).wait()

        table = table_hbm.at[cid]

        b0 = base // IB
        load_idx(b0, ibuf[0], isem[0])
        load_idx(b0 + 1, ibuf[1], isem[1])
        wait_idx(ibuf[0], isem[0])
        for b in range(NBUF):
            pltpu.async_copy(table.at[ibuf[0].at[0, b]], bufs[b], gsem[b])

        @pl.loop(0, NIB, step=2)
        def _(k0):
            for p in range(2):
                kk = k0 + p
                cur, nxt = ibuf[p], ibuf[1 - p]
                for j in range(IB):
                    b = j % NBUF
                    if j == IB - NBUF:
                        @pl.when(kk + 1 < NIB)
                        def _():
                            wait_idx(nxt, isem[1 - p])
                    pltpu.make_async_copy(table.at[cur.at[0, j]],
                                          bufs[b], gsem[b]).wait()
                    pltpu.async_copy(bufs[b], acc.at[cur.at[1, j]], ssem[b],
                                     add=True)
                    pltpu.make_async_copy(bufs[b], acc.at[cur.at[1, j]],
                                          ssem[b]).wait()
                    jn = j + NBUF
                    if jn < IB:
                        pltpu.async_copy(table.at[cur.at[0, jn]],
                                         bufs[b], gsem[b])
                    else:

                        @pl.when(kk + 1 < NIB)
                        def _(b=b, jn=jn):
                            pltpu.async_copy(table.at[nxt.at[0, jn - IB]],
                                             bufs[b], gsem[b])

                @pl.when(kk + 2 < NIB)
                def _():
                    load_idx(b0 + kk + 2, cur, isem[p])

        plsc.subcore_barrier()
        pltpu.sync_copy(acc.at[pl.ds(sid * ZPT, ZPT)],
                        out_hbm.at[cid, pl.ds(sid * ZPT, ZPT)])

    return spmm


_spmm_cache = {}


def _spmm(width):
    if width not in _spmm_cache:
        _spmm_cache[width] = _make_spmm(width)
    return _spmm_cache[width]


_MMBLK = 2000


def _mm1_body(x1_ref, x2_ref, w_ref, o_ref):
    g = pl.program_id(0)

    @pl.when(g == 0)
    def _():
        o_ref[0] = jnp.dot(x1_ref[...], w_ref[...],
                           preferred_element_type=jnp.float32)

    @pl.when(g == 1)
    def _():
        o_ref[0] = jnp.dot(x2_ref[...], w_ref[...],
                           preferred_element_type=jnp.float32)


def _mm1(x1, x2, w1):
    return pl.pallas_call(
        _mm1_body,
        grid=(NCORES, N // _MMBLK),
        in_specs=[
            pl.BlockSpec((_MMBLK, NFEAT), lambda g, i: (i, 0)),
            pl.BlockSpec((_MMBLK, NFEAT), lambda g, i: (i, 0)),
            pl.BlockSpec((NFEAT, NHID), lambda g, i: (0, 0)),
        ],
        out_specs=pl.BlockSpec((1, _MMBLK, NHID), lambda g, i: (g, i, 0)),
        out_shape=jax.ShapeDtypeStruct((NCORES, N, NHID), jnp.float32),
    )(x1, x2, w1)


def _fused2_body(p_ref, b1_ref, w2_ref, o_ref):
    a = p_ref[0] + b1_ref[...]
    h = jnp.maximum(a, 0.0)
    o_ref[0] = jnp.dot(h, w2_ref[...], preferred_element_type=jnp.float32)


def _fused2(parts, b1, w2):
    return pl.pallas_call(
        _fused2_body,
        grid=(NCORES, N // _MMBLK),
        in_specs=[
            pl.BlockSpec((1, _MMBLK, NHID), lambda g, i: (g, i, 0)),
            pl.BlockSpec((1, NHID), lambda g, i: (0, 0)),
            pl.BlockSpec((NHID, NOUT), lambda g, i: (0, 0)),
        ],
        out_specs=pl.BlockSpec((1, _MMBLK, NOUT), lambda g, i: (g, i, 0)),
        out_shape=jax.ShapeDtypeStruct((NCORES, N, NOUT), jnp.float32),
    )(parts, b1, w2)


def _final_body(p_ref, o_ref):
    d = p_ref[0] - p_ref[1] + 1e-6
    o_ref[...] = jnp.sqrt(jnp.sum(d * d, axis=1))


def _final(parts):
    return pl.pallas_call(
        _final_body,
        out_shape=jax.ShapeDtypeStruct((N,), jnp.float32),
    )(parts)


def kernel(x1, adj1, x2, adj2, W1, b1, W2, b2):
    adj1r = adj1.reshape(2, CPG, CHUNK)
    adj2r = adj2.reshape(2, CPG, CHUNK)

    support = _mm1(x1, x2, W1)
    agg1 = _spmm(NHID)(support, adj1r, adj2r)
    support2 = _fused2(agg1, b1.reshape(1, NHID), W2)
    agg2 = _spmm(NOUT)(support2, adj1r, adj2r)
    return _final(agg2)
